# Initial kernel scaffold; baseline (speedup 1.0000x reference)
import jax
import jax.numpy as jnp
from jax.experimental import pallas as pl


def kernel(gaz_seq_tensor, gaz_seq_lengths, gaz_mask_tensor, table):
    raise NotImplementedError("write your pallas kernel here")


# speedup vs baseline: 7.6682x; 7.6682x over previous
import functools

import jax
import jax.numpy as jnp
from jax import lax
from jax.experimental import pallas as pl
from jax.experimental.pallas import tpu as pltpu
from jax.experimental.pallas import tpu_sc as plsc

_B, _S, _G, _V, _D = 4096, 50, 5, 100000, 64
_N = _B * _S
_NC, _NS = 2, 16
_NW = _NC * _NS
_TPW = _N // _NW
_T = 128
_NBLK = _TPW // _T
_E = _T * _G
_IDX_ROW = 128
_NGA = _E // _IDX_ROW


def _sc_body(idx_hbm, len_hbm, table_hbm, out_hbm,
             idx_v, len_v, rows_v, out_v, isem, gsem, osem):
  wid = lax.axis_index("s") * _NC + lax.axis_index("c")
  tok0 = wid * _TPW

  def blk_body(blk, carry):
    tbase = tok0 + blk * _T
    c_idx = pltpu.make_async_copy(
        idx_hbm.at[pl.ds(tbase * _G, _E)], idx_v, isem)
    c_len = pltpu.make_async_copy(
        len_hbm.at[pl.ds(tbase, _T)], len_v, isem)
    c_idx.start()
    c_len.start()
    c_idx.wait()
    c_len.wait()

    for j in range(_NGA):
      pltpu.make_async_copy(
          table_hbm.at[idx_v.at[pl.ds(j * _IDX_ROW, _IDX_ROW)]],
          rows_v.at[pl.ds(j * _IDX_ROW, _IDX_ROW)], gsem).start()
    for j in range(_NGA):
      pltpu.make_async_copy(
          table_hbm.at[idx_v.at[pl.ds(j * _IDX_ROW, _IDX_ROW)]],
          rows_v.at[pl.ds(j * _IDX_ROW, _IDX_ROW)], gsem).wait()

    def grp_body(jj, c):
      t0 = jj * 16
      lnv = len_v[pl.ds(t0, 16)]
      ivv = 1.0 / lnv.astype(jnp.float32)
      wvs = [jnp.where(g < lnv, ivv, 0.0) for g in range(_G)]
      for k in range(16):
        r0 = (t0 + k) * _G
        accs = [jnp.zeros((16,), jnp.float32) for _ in range(4)]
        for g in range(_G):
          w = wvs[g][k]
          for cc in range(4):
            accs[cc] = accs[cc] + rows_v[r0 + g, pl.ds(cc * 16, 16)] * w
        for cc in range(4):
          out_v[t0 + k, pl.ds(cc * 16, 16)] = accs[cc]
      return c

    lax.fori_loop(0, _T // 16, grp_body, 0)

    pltpu.make_async_copy(out_v, out_hbm.at[pl.ds(tbase, _T)], osem).start()
    pltpu.make_async_copy(out_v, out_hbm.at[pl.ds(tbase, _T)], osem).wait()
    return carry

  lax.fori_loop(0, _NBLK, blk_body, 0)


_sc_call = functools.partial(
    pl.kernel,
    mesh=plsc.VectorSubcoreMesh(core_axis_name="c", subcore_axis_name="s"),
    out_type=jax.ShapeDtypeStruct((_N, _D), jnp.float32),
    compiler_params=pltpu.CompilerParams(use_tc_tiling_on_sc=False),
    scratch_types=[
        pltpu.VMEM((_E,), jnp.int32),
        pltpu.VMEM((_T,), jnp.int32),
        pltpu.VMEM((_E, _D), jnp.float32),
        pltpu.VMEM((_T, _D), jnp.float32),
        pltpu.SemaphoreType.DMA,
        pltpu.SemaphoreType.DMA,
        pltpu.SemaphoreType.DMA,
    ],
)(_sc_body)


@jax.jit
def _impl(gaz_seq_tensor, gaz_seq_lengths, table):
  idx = gaz_seq_tensor.astype(jnp.int32).reshape(_N * _G)
  lens = gaz_seq_lengths.astype(jnp.int32).reshape(_N)
  out = _sc_call(idx, lens, table)
  return out.reshape(_B, _S, _D)


def kernel(gaz_seq_tensor, gaz_seq_lengths, gaz_mask_tensor, table):
  del gaz_mask_tensor
  return _impl(gaz_seq_tensor, gaz_seq_lengths, table)


# speedup vs baseline: 9.9722x; 1.3005x over previous
import functools

import jax
import jax.numpy as jnp
from jax import lax
from jax.experimental import pallas as pl
from jax.experimental.pallas import tpu as pltpu
from jax.experimental.pallas import tpu_sc as plsc

_B, _S, _G, _V, _D = 4096, 50, 5, 100000, 64
_N = _B * _S
_NC, _NS = 2, 16
_NW = _NC * _NS
_TPW = _N // _NW
_T = 128
_NBLK = _TPW // _T
_NPAIR = _NBLK // 2
_E = _T * _G
_IDX_ROW = 128
_NGA = _E // _IDX_ROW


def _sc_body(idx_hbm, len_hbm, table_hbm, out_hbm,
             idx_v, len_v, rows_v, out_v,
             isem0, isem1, gsem0, gsem1, osem0, osem1):
  isems = (isem0, isem1)
  gsems = (gsem0, gsem1)
  osems = (osem0, osem1)
  wid = lax.axis_index("s") * _NC + lax.axis_index("c")
  tok0 = wid * _TPW

  def in_copy(blk, b):
    tb = tok0 + blk * _T
    return pltpu.make_async_copy(idx_hbm.at[pl.ds(tb * _G, _E)],
                                 idx_v.at[b], isems[b])

  def in_start(blk, b):
    in_copy(blk, b).start()

  def in_wait(blk, b):
    in_copy(blk, b).wait()

  def g_copies(b):
    return [
        pltpu.make_async_copy(
            table_hbm.at[idx_v.at[b, pl.ds(j * _IDX_ROW, _IDX_ROW)]],
            rows_v.at[b, pl.ds(j * _IDX_ROW, _IDX_ROW)], gsems[b])
        for j in range(_NGA)
    ]

  def g_start(b):
    for c in g_copies(b):
      c.start()

  def g_wait(b):
    for c in g_copies(b):
      c.wait()

  def out_copy(blk, b):
    tb = tok0 + blk * _T
    return pltpu.make_async_copy(out_v.at[b], out_hbm.at[pl.ds(tb, _T)],
                                 osems[b])

  def compute(blk, b):
    def grp_body(jj, c):
      t0 = jj * 16
      lnv = len_v[pl.ds(blk * _T + t0, 16)]
      ivv = 1.0 / lnv.astype(jnp.float32)
      wvs = [jnp.where(g < lnv, ivv, 0.0) for g in range(_G)]
      for k in range(16):
        r0 = (t0 + k) * _G
        accs = [jnp.zeros((16,), jnp.float32) for _ in range(4)]
        for g in range(_G):
          w = wvs[g][k]
          for cc in range(4):
            accs[cc] = accs[cc] + rows_v[b, r0 + g, pl.ds(cc * 16, 16)] * w
        for cc in range(4):
          out_v[b, t0 + k, pl.ds(cc * 16, 16)] = accs[cc]
      return c

    lax.fori_loop(0, _T // 16, grp_body, 0)

  lcopy = pltpu.make_async_copy(len_hbm.at[pl.ds(tok0, _TPW)], len_v,
                                isems[0])
  lcopy.start()
  in_start(0, 0)
  lcopy.wait()
  in_wait(0, 0)
  g_start(0)
  in_start(1, 1)

  def pair_body(o, carry):
    for b in range(2):
      blk = o * 2 + b
      g_wait(b)
      pl.when(o < _NPAIR - 1)(functools.partial(in_start, blk + 2, b))
      if b == 0:
        in_wait(blk + 1, 1 - b)
        g_start(1 - b)
      else:
        def _fire():
          in_wait(blk + 1, 1 - b)
          g_start(1 - b)
        pl.when(o < _NPAIR - 1)(_fire)
      pl.when(o > 0)(lambda: out_copy(blk - 2, b).wait())
      compute(blk, b)
      out_copy(blk, b).start()
    return carry

  lax.fori_loop(0, _NPAIR, pair_body, 0)

  out_copy(_NBLK - 2, 0).wait()
  out_copy(_NBLK - 1, 1).wait()


_sc_call = functools.partial(
    pl.kernel,
    mesh=plsc.VectorSubcoreMesh(core_axis_name="c", subcore_axis_name="s"),
    out_type=jax.ShapeDtypeStruct((_N, _D), jnp.float32),
    compiler_params=pltpu.CompilerParams(use_tc_tiling_on_sc=False),
    scratch_types=[
        pltpu.VMEM((2, _E), jnp.int32),
        pltpu.VMEM((_TPW,), jnp.int32),
        pltpu.VMEM((2, _E, _D), jnp.float32),
        pltpu.VMEM((2, _T, _D), jnp.float32),
        pltpu.SemaphoreType.DMA,
        pltpu.SemaphoreType.DMA,
        pltpu.SemaphoreType.DMA,
        pltpu.SemaphoreType.DMA,
        pltpu.SemaphoreType.DMA,
        pltpu.SemaphoreType.DMA,
    ],
)(_sc_body)


@jax.jit
def _impl(gaz_seq_tensor, gaz_seq_lengths, table):
  idx = gaz_seq_tensor.astype(jnp.int32).reshape(_N * _G)
  lens = gaz_seq_lengths.astype(jnp.int32).reshape(_N)
  out = _sc_call(idx, lens, table)
  return out.reshape(_B, _S, _D)


def kernel(gaz_seq_tensor, gaz_seq_lengths, gaz_mask_tensor, table):
  del gaz_mask_tensor
  return _impl(gaz_seq_tensor, gaz_seq_lengths, table)


# speedup vs baseline: 14.8820x; 1.4923x over previous
import functools

import jax
import jax.numpy as jnp
from jax import lax
from jax.experimental import pallas as pl
from jax.experimental.pallas import tpu as pltpu
from jax.experimental.pallas import tpu_sc as plsc

_B, _S, _G, _V, _D = 4096, 50, 5, 100000, 64
_N = _B * _S
_NC, _NS = 2, 16
_NW = _NC * _NS
_BW = _B // _NW
_T = _BW
_NBLK = _S
_NPAIR = _NBLK // 2
_E = _T * _G
_DH = _D // 8


def _sc_body(idx_hbm, len_hbm, table_hbm, out_hbm,
             idx_v, len_v, rows_v, out_v,
             isem0, isem1, gsem0, gsem1, osem0, osem1):
  isems = (isem0, isem1)
  gsems = (gsem0, gsem1)
  osems = (osem0, osem1)
  wid = lax.axis_index("s") * _NC + lax.axis_index("c")
  b0 = wid * _BW

  def idx_copy(s, b):
    return pltpu.make_async_copy(idx_hbm.at[:, s, pl.ds(b0, _BW)],
                                 idx_v.at[b], isems[b])

  def len_copy(s, b):
    return pltpu.make_async_copy(len_hbm.at[s, pl.ds(b0, _BW)],
                                 len_v.at[b], isems[b])

  def in_start(s, b):
    idx_copy(s, b).start()
    len_copy(s, b).start()

  def in_wait(s, b):
    idx_copy(s, b).wait()
    len_copy(s, b).wait()

  def g_copies(b):
    return [
        pltpu.make_async_copy(
            table_hbm.at[idx_v.at[b, g]],
            rows_v.at[b, pl.ds(g * _T, _T)], gsems[b])
        for g in range(_G)
    ]

  def g_start(b):
    for c in g_copies(b):
      c.start()

  def g_wait(b):
    for c in g_copies(b):
      c.wait()

  def out_copy(s, b):
    return pltpu.make_async_copy(
        out_v.at[b], out_hbm.at[s, pl.ds(b0, _BW), :], osems[b])

  def compute(b):
    def grp_body(jj, c):
      t0 = jj * 16
      lnv = len_v[b, pl.ds(t0, 16)]
      ivv = 1.0 / lnv.astype(jnp.float32)
      wvs = [jnp.where(g < lnv, ivv, 0.0) for g in range(_G)]
      for k in range(16):
        t = t0 + k
        accs = [jnp.zeros((16,), jnp.float32) for _ in range(4)]
        for g in range(_G):
          w = wvs[g][k]
          for cc in range(4):
            accs[cc] = accs[cc] + rows_v[b, g * _T + t, pl.ds(cc * 16, 16)] * w
        for cc in range(4):
          out_v[b, t, pl.ds(cc * 16, 16)] = accs[cc]
      return c

    lax.fori_loop(0, _T // 16, grp_body, 0)

  in_start(0, 0)
  in_wait(0, 0)
  g_start(0)
  in_start(1, 1)

  def pair_body(o, carry):
    for b in range(2):
      s = o * 2 + b
      g_wait(b)
      pl.when(o < _NPAIR - 1)(lambda: idx_copy(s + 2, b).start())
      if b == 0:
        in_wait(s + 1, 1 - b)
        g_start(1 - b)
      else:
        def _fire():
          in_wait(s + 1, 1 - b)
          g_start(1 - b)
        pl.when(o < _NPAIR - 1)(_fire)
      pl.when(o > 0)(lambda: out_copy(s - 2, b).wait())
      compute(b)
      pl.when(o < _NPAIR - 1)(lambda: len_copy(s + 2, b).start())
      out_copy(s, b).start()
    return carry

  lax.fori_loop(0, _NPAIR, pair_body, 0)

  out_copy(_NBLK - 2, 0).wait()
  out_copy(_NBLK - 1, 1).wait()


_sc_call = functools.partial(
    pl.kernel,
    mesh=plsc.VectorSubcoreMesh(core_axis_name="c", subcore_axis_name="s"),
    out_type=jax.ShapeDtypeStruct((_S, _B, _D), jnp.float32),
    compiler_params=pltpu.CompilerParams(use_tc_tiling_on_sc=False),
    scratch_types=[
        pltpu.VMEM((2, _G, _BW), jnp.int32),
        pltpu.VMEM((2, _BW), jnp.int32),
        pltpu.VMEM((2, _E, _D), jnp.float32),
        pltpu.VMEM((2, _BW, _D), jnp.float32),
        pltpu.SemaphoreType.DMA,
        pltpu.SemaphoreType.DMA,
        pltpu.SemaphoreType.DMA,
        pltpu.SemaphoreType.DMA,
        pltpu.SemaphoreType.DMA,
        pltpu.SemaphoreType.DMA,
    ],
)(_sc_body)


@jax.jit
def _impl(gaz_seq_tensor, gaz_seq_lengths, table):
  idx = gaz_seq_tensor.astype(jnp.int32).transpose(2, 1, 0)
  lens = gaz_seq_lengths.astype(jnp.int32).transpose(1, 0)
  out = _sc_call(idx, lens, table)
  return out.transpose(1, 0, 2)


def kernel(gaz_seq_tensor, gaz_seq_lengths, gaz_mask_tensor, table):
  del gaz_mask_tensor
  return _impl(gaz_seq_tensor, gaz_seq_lengths, table)


# speedup vs baseline: 18.8621x; 1.2674x over previous
import functools

import jax
import jax.numpy as jnp
from jax import lax
from jax.experimental import pallas as pl
from jax.experimental.pallas import tpu as pltpu
from jax.experimental.pallas import tpu_sc as plsc

_B, _S, _G, _V, _D = 4096, 50, 5, 100000, 64
_N = _B * _S
_NC, _NS = 2, 16
_NW = _NC * _NS
_BW = _B // _NW
_T = _BW
_NBLK = _S
_NPAIR = _NBLK // 2
_E = _T * _G
_DH = _D // 8


def _sc_body(idx_hbm, len_hbm, table_hbm, out_hbm,
             idx_v, len_v, rows_v, out_v,
             isem0, isem1, gsem0, gsem1, osem0, osem1):
  isems = (isem0, isem1)
  gsems = (gsem0, gsem1)
  osems = (osem0, osem1)
  wid = lax.axis_index("s") * _NC + lax.axis_index("c")
  b0 = wid * _BW

  def idx_copy(s, b):
    return pltpu.make_async_copy(idx_hbm.at[:, s, pl.ds(b0, _BW)],
                                 idx_v.at[b], isems[b])

  def len_copy(s, b):
    return pltpu.make_async_copy(len_hbm.at[s, pl.ds(b0, _BW)],
                                 len_v.at[b], isems[b])

  def in_start(s, b):
    idx_copy(s, b).start()
    len_copy(s, b).start()

  def in_wait(s, b):
    idx_copy(s, b).wait()
    len_copy(s, b).wait()

  def g_copies(b):
    return [
        pltpu.make_async_copy(
            table_hbm.at[idx_v.at[b, g]],
            rows_v.at[b, pl.ds(g * _T, _T)], gsems[b])
        for g in range(_G)
    ]

  def g_start(b):
    for c in g_copies(b):
      c.start()

  def g_wait(b):
    for c in g_copies(b):
      c.wait()

  def out_copy(s, b):
    return pltpu.make_async_copy(
        out_v.at[b],
        out_hbm.at[pl.ds(b0, _BW), s // 8, s % 8, pl.ds(0, _D)], osems[b])

  def compute(b):
    def grp_body(jj, c):
      t0 = jj * 16
      lnv = len_v[b, pl.ds(t0, 16)]
      ivv = 1.0 / lnv.astype(jnp.float32)
      wvs = [jnp.where(g < lnv, ivv, 0.0) for g in range(_G)]
      for k in range(16):
        t = t0 + k
        accs = [jnp.zeros((16,), jnp.float32) for _ in range(4)]
        for g in range(_G):
          w = wvs[g][k]
          for cc in range(4):
            accs[cc] = accs[cc] + rows_v[b, g * _T + t, pl.ds(cc * 16, 16)] * w
        for cc in range(4):
          out_v[b, t, pl.ds(cc * 16, 16)] = accs[cc]
      return c

    lax.fori_loop(0, _T // 16, grp_body, 0)

  in_start(0, 0)
  in_wait(0, 0)
  g_start(0)
  in_start(1, 1)

  def pair_body(o, carry):
    for b in range(2):
      s = o * 2 + b
      g_wait(b)
      pl.when(o < _NPAIR - 1)(lambda: idx_copy(s + 2, b).start())
      if b == 0:
        in_wait(s + 1, 1 - b)
        g_start(1 - b)
      else:
        def _fire():
          in_wait(s + 1, 1 - b)
          g_start(1 - b)
        pl.when(o < _NPAIR - 1)(_fire)
      pl.when(o > 0)(lambda: out_copy(s - 2, b).wait())
      compute(b)
      pl.when(o < _NPAIR - 1)(lambda: len_copy(s + 2, b).start())
      out_copy(s, b).start()
    return carry

  lax.fori_loop(0, _NPAIR, pair_body, 0)

  out_copy(_NBLK - 2, 0).wait()
  out_copy(_NBLK - 1, 1).wait()


_sc_call = functools.partial(
    pl.kernel,
    mesh=plsc.VectorSubcoreMesh(core_axis_name="c", subcore_axis_name="s"),
    out_type=jax.ShapeDtypeStruct((_B, 7, 8, 128), jnp.float32),
    compiler_params=pltpu.CompilerParams(use_tc_tiling_on_sc=False),
    scratch_types=[
        pltpu.VMEM((2, _G, _BW), jnp.int32),
        pltpu.VMEM((2, _BW), jnp.int32),
        pltpu.VMEM((2, _E, _D), jnp.float32),
        pltpu.VMEM((2, _BW, _D), jnp.float32),
        pltpu.SemaphoreType.DMA,
        pltpu.SemaphoreType.DMA,
        pltpu.SemaphoreType.DMA,
        pltpu.SemaphoreType.DMA,
        pltpu.SemaphoreType.DMA,
        pltpu.SemaphoreType.DMA,
    ],
)(_sc_body)


@jax.jit
def _impl(gaz_seq_tensor, gaz_seq_lengths, table):
  idx = gaz_seq_tensor.astype(jnp.int32).transpose(2, 1, 0)
  lens = gaz_seq_lengths.astype(jnp.int32).transpose(1, 0)
  out = _sc_call(idx, lens, table)
  return out.reshape(_B, 56, 128)[:, :_S, :_D]


def kernel(gaz_seq_tensor, gaz_seq_lengths, gaz_mask_tensor, table):
  del gaz_mask_tensor
  return _impl(gaz_seq_tensor, gaz_seq_lengths, table)


# speedup vs baseline: 18.8939x; 1.0017x over previous
import functools

import jax
import jax.numpy as jnp
from jax import lax
from jax.experimental import pallas as pl
from jax.experimental.pallas import tpu as pltpu
from jax.experimental.pallas import tpu_sc as plsc

_B, _S, _G, _V, _D = 4096, 50, 5, 100000, 64
_N = _B * _S
_NC, _NS = 2, 16
_NW = _NC * _NS
_BW = _B // _NW
_T = _BW
_NBLK = _S
_NPAIR = _NBLK // 2
_E = _T * _G
_DH = _D // 8


def _sc_body(idx_hbm, len_hbm, table_hbm, out_hbm,
             idx_v, len_v, rows_v, out_v,
             isem0, isem1, gsem0, gsem1, osem0, osem1):
  isems = (isem0, isem1)
  gsems = (gsem0, gsem1)
  osems = (osem0, osem1)
  wid = lax.axis_index("s") * _NC + lax.axis_index("c")
  b0 = wid * _BW

  def idx_copy(s, b):
    return pltpu.make_async_copy(
        idx_hbm.at[:, s // 8, wid, pl.ds((s % 8) * _BW, _BW)],
        idx_v.at[b], isems[b])

  def len_copy(s, b):
    return pltpu.make_async_copy(
        len_hbm.at[s // 8, wid, pl.ds((s % 8) * _BW, _BW)],
        len_v.at[b], isems[b])

  def in_start(s, b):
    idx_copy(s, b).start()
    len_copy(s, b).start()

  def in_wait(s, b):
    idx_copy(s, b).wait()
    len_copy(s, b).wait()

  def g_copies(b):
    return [
        pltpu.make_async_copy(
            table_hbm.at[idx_v.at[b, g]],
            rows_v.at[b, pl.ds(g * _T, _T)], gsems[b])
        for g in range(_G)
    ]

  def g_start(b):
    for c in g_copies(b):
      c.start()

  def g_wait(b):
    for c in g_copies(b):
      c.wait()

  def out_copy(s, b):
    return pltpu.make_async_copy(
        out_v.at[b],
        out_hbm.at[pl.ds(b0, _BW), s // 8, s % 8, pl.ds(0, _D)], osems[b])

  def compute(b):
    def grp_body(jj, c):
      t0 = jj * 16
      lnv = len_v[b, pl.ds(t0, 16)]
      ivv = 1.0 / lnv.astype(jnp.float32)
      wvs = [jnp.where(g < lnv, ivv, 0.0) for g in range(_G)]
      for k in range(16):
        t = t0 + k
        accs = [jnp.zeros((16,), jnp.float32) for _ in range(4)]
        for g in range(_G):
          w = wvs[g][k]
          for cc in range(4):
            accs[cc] = accs[cc] + rows_v[b, g * _T + t, pl.ds(cc * 16, 16)] * w
        for cc in range(4):
          out_v[b, t, pl.ds(cc * 16, 16)] = accs[cc]
      return c

    lax.fori_loop(0, _T // 16, grp_body, 0)

  in_start(0, 0)
  in_wait(0, 0)
  g_start(0)
  in_start(1, 1)

  def pair_body(o, carry):
    for b in range(2):
      s = o * 2 + b
      g_wait(b)
      pl.when(o < _NPAIR - 1)(lambda: idx_copy(s + 2, b).start())
      if b == 0:
        in_wait(s + 1, 1 - b)
        g_start(1 - b)
      else:
        def _fire():
          in_wait(s + 1, 1 - b)
          g_start(1 - b)
        pl.when(o < _NPAIR - 1)(_fire)
      pl.when(o > 0)(lambda: out_copy(s - 2, b).wait())
      compute(b)
      pl.when(o < _NPAIR - 1)(lambda: len_copy(s + 2, b).start())
      out_copy(s, b).start()
    return carry

  lax.fori_loop(0, _NPAIR, pair_body, 0)

  out_copy(_NBLK - 2, 0).wait()
  out_copy(_NBLK - 1, 1).wait()


_sc_call = functools.partial(
    pl.kernel,
    mesh=plsc.VectorSubcoreMesh(core_axis_name="c", subcore_axis_name="s"),
    out_type=jax.ShapeDtypeStruct((_B, 7, 8, 128), jnp.float32),
    compiler_params=pltpu.CompilerParams(use_tc_tiling_on_sc=False),
    scratch_types=[
        pltpu.VMEM((2, _G, _BW), jnp.int32),
        pltpu.VMEM((2, _BW), jnp.int32),
        pltpu.VMEM((2, _E, _D), jnp.float32),
        pltpu.VMEM((2, _BW, _D), jnp.float32),
        pltpu.SemaphoreType.DMA,
        pltpu.SemaphoreType.DMA,
        pltpu.SemaphoreType.DMA,
        pltpu.SemaphoreType.DMA,
        pltpu.SemaphoreType.DMA,
        pltpu.SemaphoreType.DMA,
    ],
)(_sc_body)


@jax.jit
def _impl(gaz_seq_tensor, gaz_seq_lengths, table):
  idx = gaz_seq_tensor.astype(jnp.int32).transpose(2, 1, 0)
  idx = jnp.pad(idx, ((0, 0), (0, 6), (0, 0)))
  idx = idx.reshape(_G, 7, 8, _NW, _BW).transpose(0, 1, 3, 2, 4)
  idx = idx.reshape(_G, 7, _NW, 8 * _BW)
  lens = gaz_seq_lengths.astype(jnp.int32).transpose(1, 0)
  lens = jnp.pad(lens, ((0, 6), (0, 0)))
  lens = lens.reshape(7, 8, _NW, _BW).transpose(0, 2, 1, 3)
  lens = lens.reshape(7, _NW, 8 * _BW)
  out = _sc_call(idx, lens, table)
  return out.reshape(_B, 56, 128)[:, :_S, :_D]


def kernel(gaz_seq_tensor, gaz_seq_lengths, gaz_mask_tensor, table):
  del gaz_mask_tensor
  return _impl(gaz_seq_tensor, gaz_seq_lengths, table)
